# Initial kernel scaffold; baseline (speedup 1.0000x reference)
import jax
import jax.numpy as jnp
from jax.experimental import pallas as pl


def kernel(x, edge_index, W1, b1, W2, b2):
    raise NotImplementedError("write your pallas kernel here")


# speedup vs baseline: 15.3525x; 15.3525x over previous
import functools

import jax
import jax.numpy as jnp
from jax import lax
from jax.experimental import pallas as pl
from jax.experimental.pallas import tpu as pltpu
from jax.experimental.pallas import tpu_sc as plsc

N_NODES = 10000
N_EDGES = 320000
D = 128

NC = 2
NS = 16
NW = NC * NS

K = 128
N_PAD = 10240
ROWS_PER_TILE = N_PAD // NS

CHUNKS_PER_WORKER = -(-N_EDGES // (K * NW))
E_PAD = CHUNKS_PER_WORKER * K * NW
N_CHUNKS = E_PAD // K


def _make_sc_agg(width):
    mesh = plsc.VectorSubcoreMesh(core_axis_name="c", subcore_axis_name="s")

    @functools.partial(
        pl.kernel,
        out_type=jax.ShapeDtypeStruct((NC, N_PAD, width), jnp.float32),
        mesh=mesh,
        scratch_types=[
            pltpu.VMEM((K,), jnp.int32),
            pltpu.VMEM((K,), jnp.int32),
            pltpu.VMEM((K, width), jnp.float32),
            pltpu.VMEM((ROWS_PER_TILE // 10, width), jnp.float32),
            pltpu.VMEM_SHARED((N_PAD, width), jnp.float32),
            pltpu.SemaphoreType.DMA,
        ],
    )
    def sc_agg(y_hbm, src_hbm, dst_hbm, out_hbm,
               src_v, dst_v, rows_v, zero_v, acc_sh, sem):
        cid = lax.axis_index("c")
        sid = lax.axis_index("s")
        wid = sid * NC + cid

        zb = zero_v.shape[0]
        def zloop(i, _):
            zero_v[i // (width // 16), pl.ds((i % (width // 16)) * 16, 16)] = (
                jnp.zeros((16,), jnp.float32))
            return 0
        lax.fori_loop(0, zb * (width // 16), zloop, 0)
        def zcopy(j, _):
            pltpu.sync_copy(zero_v, acc_sh.at[pl.ds(sid * ROWS_PER_TILE + j * zb, zb)])
            return 0
        lax.fori_loop(0, ROWS_PER_TILE // zb, zcopy, 0)
        plsc.subcore_barrier()

        def body(c, _):
            chunk = wid * CHUNKS_PER_WORKER + c
            pltpu.sync_copy(src_hbm.at[chunk], src_v)
            pltpu.sync_copy(dst_hbm.at[chunk], dst_v)
            pltpu.async_copy(y_hbm.at[src_v], rows_v, sem).wait()
            pltpu.sync_copy(rows_v, acc_sh.at[dst_v], add=True)
            return 0
        lax.fori_loop(0, CHUNKS_PER_WORKER, body, 0)
        plsc.subcore_barrier()

        pltpu.sync_copy(
            acc_sh.at[pl.ds(sid * ROWS_PER_TILE, ROWS_PER_TILE)],
            out_hbm.at[cid, pl.ds(sid * ROWS_PER_TILE, ROWS_PER_TILE)],
        )

    return sc_agg


def _make_sc_hist():
    mesh = plsc.VectorSubcoreMesh(core_axis_name="c", subcore_axis_name="s")
    HW = 16

    @functools.partial(
        pl.kernel,
        out_type=jax.ShapeDtypeStruct((NC, N_PAD, HW), jnp.float32),
        mesh=mesh,
        scratch_types=[
            pltpu.VMEM((K,), jnp.int32),
            pltpu.VMEM((K, HW), jnp.float32),
            pltpu.VMEM_SHARED((N_PAD, HW), jnp.float32),
        ],
    )
    def sc_hist(dst_hbm, out_hbm, dst_v, ones_v, acc_sh):
        cid = lax.axis_index("c")
        sid = lax.axis_index("s")
        wid = sid * NC + cid

        def fill(val):
            def f(i, _):
                ones_v[i, :] = jnp.full((16,), val, jnp.float32)
                return 0
            lax.fori_loop(0, K, f, 0)
        fill(0.0)
        def zcopy(j, _):
            pltpu.sync_copy(ones_v, acc_sh.at[pl.ds(sid * ROWS_PER_TILE + j * K, K)])
            return 0
        lax.fori_loop(0, ROWS_PER_TILE // K, zcopy, 0)
        fill(1.0)
        plsc.subcore_barrier()

        def body(c, _):
            chunk = wid * CHUNKS_PER_WORKER + c
            pltpu.sync_copy(dst_hbm.at[chunk], dst_v)
            pltpu.sync_copy(ones_v, acc_sh.at[dst_v], add=True)
            return 0
        lax.fori_loop(0, CHUNKS_PER_WORKER, body, 0)
        plsc.subcore_barrier()

        pltpu.sync_copy(
            acc_sh.at[pl.ds(sid * ROWS_PER_TILE, ROWS_PER_TILE)],
            out_hbm.at[cid, pl.ds(sid * ROWS_PER_TILE, ROWS_PER_TILE)],
        )

    return sc_hist


_BLK = 512


def _tc_scale_matmul_kernel(deg_ref, x_ref, w_ref, o_ref):
    dinv = lax.rsqrt(deg_ref[...])
    o_ref[...] = dinv * jnp.dot(x_ref[...], w_ref[...],
                                preferred_element_type=jnp.float32)


def _tc_scale_matmul(deg, x, w):
    n = x.shape[0]
    grid = (n // _BLK,)
    return pl.pallas_call(
        _tc_scale_matmul_kernel,
        grid=grid,
        in_specs=[
            pl.BlockSpec((_BLK, 1), lambda i: (i, 0)),
            pl.BlockSpec((_BLK, D), lambda i: (i, 0)),
            pl.BlockSpec((D, D), lambda i: (0, 0)),
        ],
        out_specs=pl.BlockSpec((_BLK, D), lambda i: (i, 0)),
        out_shape=jax.ShapeDtypeStruct((n, D), jnp.float32),
    )(deg, x, w)


def _tc_mid_kernel(deg_ref, p_ref, y_ref, b_ref, w_ref, o_ref):
    dinv = lax.rsqrt(deg_ref[...])
    h = jax.nn.relu(dinv * (p_ref[0] + p_ref[1] + y_ref[...]) + b_ref[...])
    o_ref[...] = dinv * jnp.dot(h, w_ref[...], preferred_element_type=jnp.float32)


def _tc_mid(deg, parts, y, b, w):
    n = y.shape[0]
    grid = (n // _BLK,)
    return pl.pallas_call(
        _tc_mid_kernel,
        grid=grid,
        in_specs=[
            pl.BlockSpec((_BLK, 1), lambda i: (i, 0)),
            pl.BlockSpec((NC, _BLK, D), lambda i: (0, i, 0)),
            pl.BlockSpec((_BLK, D), lambda i: (i, 0)),
            pl.BlockSpec((1, D), lambda i: (0, 0)),
            pl.BlockSpec((D, D), lambda i: (0, 0)),
        ],
        out_specs=pl.BlockSpec((_BLK, D), lambda i: (i, 0)),
        out_shape=jax.ShapeDtypeStruct((n, D), jnp.float32),
    )(deg, parts, y, b, w)


def _tc_final_kernel(deg_ref, p_ref, y_ref, b_ref, o_ref):
    dinv = lax.rsqrt(deg_ref[...])
    o_ref[...] = jax.nn.relu(dinv * (p_ref[0] + p_ref[1] + y_ref[...]) + b_ref[...])


def _tc_final(deg, parts, y, b):
    n = y.shape[0]
    grid = (n // _BLK,)
    return pl.pallas_call(
        _tc_final_kernel,
        grid=grid,
        in_specs=[
            pl.BlockSpec((_BLK, 1), lambda i: (i, 0)),
            pl.BlockSpec((NC, _BLK, D), lambda i: (0, i, 0)),
            pl.BlockSpec((_BLK, D), lambda i: (i, 0)),
            pl.BlockSpec((1, D), lambda i: (0, 0)),
        ],
        out_specs=pl.BlockSpec((_BLK, D), lambda i: (i, 0)),
        out_shape=jax.ShapeDtypeStruct((n, D), jnp.float32),
    )(deg, parts, y, b)


def _tc_deg_kernel(h_ref, o_ref):
    o_ref[...] = h_ref[0, :, 0:1] + h_ref[1, :, 0:1] + 1.0


def _tc_deg(hist):
    n = hist.shape[1]
    grid = (n // _BLK,)
    return pl.pallas_call(
        _tc_deg_kernel,
        grid=grid,
        in_specs=[pl.BlockSpec((NC, _BLK, 16), lambda i: (0, i, 0))],
        out_specs=pl.BlockSpec((_BLK, 1), lambda i: (i, 0)),
        out_shape=jax.ShapeDtypeStruct((n, 1), jnp.float32),
    )(hist)


def kernel(x, edge_index, W1, b1, W2, b2):
    src = edge_index[0].astype(jnp.int32)
    dst = edge_index[1].astype(jnp.int32)

    pad = E_PAD - N_EDGES
    pad_i = jnp.arange(pad, dtype=jnp.int32)
    src_p = jnp.concatenate([src, pad_i % 16]).reshape(N_CHUNKS, K)
    dst_p = jnp.concatenate([dst, N_NODES + (pad_i % 16)]).reshape(N_CHUNKS, K)

    x_p = jnp.zeros((N_PAD, D), jnp.float32).at[:N_NODES].set(x)

    sc_hist = _make_sc_hist()
    sc_agg = _make_sc_agg(D)

    hist = sc_hist(dst_p)
    deg = _tc_deg(hist)

    y1 = _tc_scale_matmul(deg, x_p, W1)
    p1 = sc_agg(y1, src_p, dst_p)
    y2 = _tc_mid(deg, p1, y1, b1.reshape(1, D), W2)
    p2 = sc_agg(y2, src_p, dst_p)
    out = _tc_final(deg, p2, y2, b2.reshape(1, D))
    return out[:N_NODES]


# speedup vs baseline: 24.3677x; 1.5872x over previous
import functools

import jax
import jax.numpy as jnp
from jax import lax
from jax.experimental import pallas as pl
from jax.experimental.pallas import tpu as pltpu
from jax.experimental.pallas import tpu_sc as plsc

N_NODES = 10000
N_EDGES = 320000
D = 128

NC = 2
NS = 16
NW = NC * NS

K = 128
N_PAD = 10240
ROWS_PER_TILE = N_PAD // NS

CHUNKS_PER_WORKER = 80
E_PAD = CHUNKS_PER_WORKER * K * NW
N_CHUNKS = E_PAD // K


def _make_sc_agg(width):
    mesh = plsc.VectorSubcoreMesh(core_axis_name="c", subcore_axis_name="s")

    @functools.partial(
        pl.kernel,
        out_type=jax.ShapeDtypeStruct((NC, N_PAD, width), jnp.float32),
        mesh=mesh,
        scratch_types=[
            pltpu.VMEM((K,), jnp.int32),
            pltpu.VMEM((K,), jnp.int32),
            pltpu.VMEM((K,), jnp.int32),
            pltpu.VMEM((K,), jnp.int32),
            pltpu.VMEM((K,), jnp.int32),
            pltpu.VMEM((K,), jnp.int32),
            pltpu.VMEM((K,), jnp.int32),
            pltpu.VMEM((K,), jnp.int32),
            pltpu.VMEM((K, width), jnp.float32),
            pltpu.VMEM((K, width), jnp.float32),
            pltpu.VMEM_SHARED((N_PAD, width), jnp.float32),
            pltpu.SemaphoreType.DMA,
            pltpu.SemaphoreType.DMA,
            pltpu.SemaphoreType.DMA,
            pltpu.SemaphoreType.DMA,
            pltpu.SemaphoreType.DMA,
            pltpu.SemaphoreType.DMA,
            pltpu.SemaphoreType.DMA,
            pltpu.SemaphoreType.DMA,
        ],
    )
    def sc_agg(y_hbm, src_hbm, dst_hbm, out_hbm,
               sb0, sb1, sb2, sb3, db0, db1, db2, db3,
               rows0, rows1, acc_sh,
               si0, si1, si2, si3, sg0, sg1, ss0, ss1):
        cid = lax.axis_index("c")
        sid = lax.axis_index("s")
        wid = sid * NC + cid
        n = CHUNKS_PER_WORKER
        srcb = (sb0, sb1, sb2, sb3)
        dstb = (db0, db1, db2, db3)
        rows = (rows0, rows1)
        si = (si0, si1, si2, si3)
        sg = (sg0, sg1)
        ss = (ss0, ss1)

        def zloop(i, _):
            rows0[i // (width // 16), pl.ds((i % (width // 16)) * 16, 16)] = (
                jnp.zeros((16,), jnp.float32))
            return 0
        lax.fori_loop(0, K * (width // 16), zloop, 0)
        def zcopy(j, _):
            pltpu.sync_copy(rows0, acc_sh.at[pl.ds(sid * ROWS_PER_TILE + j * K, K)])
            return 0
        lax.fori_loop(0, ROWS_PER_TILE // K, zcopy, 0)
        plsc.subcore_barrier()

        def idesc_s(c, b4):
            return pltpu.make_async_copy(src_hbm.at[wid * n + c], srcb[b4],
                                         si[b4])

        def idesc_d(c, b4):
            return pltpu.make_async_copy(dst_hbm.at[wid * n + c], dstb[b4],
                                         si[b4])

        def istart(c, b4):
            pltpu.async_copy(src_hbm.at[wid * n + c], srcb[b4], si[b4])
            pltpu.async_copy(dst_hbm.at[wid * n + c], dstb[b4], si[b4])

        def iwait(c, b4):
            idesc_s(c, b4).wait()
            idesc_d(c, b4).wait()

        def gstart(b4, b2):
            pltpu.async_copy(y_hbm.at[srcb[b4]], rows[b2], sg[b2])

        def gdesc(b4, b2):
            return pltpu.make_async_copy(y_hbm.at[srcb[b4]], rows[b2], sg[b2])

        def sdesc(b4, b2):
            return pltpu.make_async_copy(rows[b2], acc_sh.at[dstb[b4]],
                                         ss[b2])

        def sstart(b4, b2):
            pltpu.async_copy(rows[b2], acc_sh.at[dstb[b4]], ss[b2], add=True)

        def step(c, b4, b2):
            gdesc(b4, b2).wait()
            sdesc((b4 + 3) % 4, 1 - b2).wait()
            c1 = jnp.minimum(c + 1, n - 1)
            iwait(c1, (b4 + 1) % 4)
            gstart((b4 + 1) % 4, 1 - b2)
            sstart(b4, b2)
            c2 = jnp.minimum(c + 2, n - 1)
            istart(c2, (b4 + 2) % 4)

        istart(0, 0)
        istart(1, 1)
        iota16 = lax.iota(jnp.int32, 16)
        for j in range(K // 16):
            dstb[3][pl.ds(16 * j, 16)] = N_NODES + 16 + 16 * j + iota16
        iwait(0, 0)
        gstart(0, 0)
        sstart(3, 1)

        def body(o, _):
            c = 4 * o
            step(c, 0, 0)
            step(c + 1, 1, 1)
            step(c + 2, 2, 0)
            step(c + 3, 3, 1)
            return 0
        lax.fori_loop(0, n // 4, body, 0)

        gdesc(0, 0).wait()
        iwait(n - 1, 1)
        sdesc(3, 1).wait()
        plsc.subcore_barrier()

        pltpu.sync_copy(
            acc_sh.at[pl.ds(sid * ROWS_PER_TILE, ROWS_PER_TILE)],
            out_hbm.at[cid, pl.ds(sid * ROWS_PER_TILE, ROWS_PER_TILE)],
        )

    return sc_agg


def _make_sc_hist():
    mesh = plsc.VectorSubcoreMesh(core_axis_name="c", subcore_axis_name="s")
    HW = 16

    LAG = 8

    @functools.partial(
        pl.kernel,
        out_type=jax.ShapeDtypeStruct((NC, N_PAD, HW), jnp.float32),
        mesh=mesh,
        scratch_types=[
            pltpu.VMEM((CHUNKS_PER_WORKER, K), jnp.int32),
            pltpu.VMEM((K, HW), jnp.float32),
            pltpu.VMEM_SHARED((N_PAD, HW), jnp.float32),
            pltpu.SemaphoreType.DMA,
            pltpu.SemaphoreType.DMA,
        ],
    )
    def sc_hist(dst_hbm, out_hbm, dst_v, ones_v, acc_sh, sem, sem2):
        cid = lax.axis_index("c")
        sid = lax.axis_index("s")
        wid = sid * NC + cid
        n = CHUNKS_PER_WORKER

        pltpu.sync_copy(dst_hbm.at[pl.ds(wid * n, n)], dst_v)

        def fill(val):
            def f(i, _):
                ones_v[i, :] = jnp.full((16,), val, jnp.float32)
                return 0
            lax.fori_loop(0, K, f, 0)
        fill(0.0)
        def zcopy(j, _):
            pltpu.sync_copy(ones_v, acc_sh.at[pl.ds(sid * ROWS_PER_TILE + j * K, K)])
            return 0
        lax.fori_loop(0, ROWS_PER_TILE // K, zcopy, 0)
        fill(1.0)
        plsc.subcore_barrier()

        def body(o, _):
            c = 2 * o
            pltpu.async_copy(ones_v, acc_sh.at[dst_v.at[c]], sem, add=True)
            pltpu.async_copy(ones_v, acc_sh.at[dst_v.at[c + 1]], sem2, add=True)
            pltpu.make_async_copy(ones_v, acc_sh.at[dst_v.at[c]], sem).wait()
            pltpu.make_async_copy(ones_v, acc_sh.at[dst_v.at[c + 1]], sem2).wait()
            return 0
        lax.fori_loop(0, n // 2, body, 0)
        plsc.subcore_barrier()

        pltpu.sync_copy(
            acc_sh.at[pl.ds(sid * ROWS_PER_TILE, ROWS_PER_TILE)],
            out_hbm.at[cid, pl.ds(sid * ROWS_PER_TILE, ROWS_PER_TILE)],
        )

    return sc_hist


_BLK = 512


def _tc_scale_matmul_kernel(deg_ref, x_ref, w_ref, o_ref):
    dinv = lax.rsqrt(deg_ref[...])
    o_ref[...] = dinv * jnp.dot(x_ref[...], w_ref[...],
                                preferred_element_type=jnp.float32)


def _tc_scale_matmul(deg, x, w):
    n = x.shape[0]
    grid = (n // _BLK,)
    return pl.pallas_call(
        _tc_scale_matmul_kernel,
        grid=grid,
        in_specs=[
            pl.BlockSpec((_BLK, 1), lambda i: (i, 0)),
            pl.BlockSpec((_BLK, D), lambda i: (i, 0)),
            pl.BlockSpec((D, D), lambda i: (0, 0)),
        ],
        out_specs=pl.BlockSpec((_BLK, D), lambda i: (i, 0)),
        out_shape=jax.ShapeDtypeStruct((n, D), jnp.float32),
    )(deg, x, w)


def _tc_mid_kernel(deg_ref, p_ref, y_ref, b_ref, w_ref, o_ref):
    dinv = lax.rsqrt(deg_ref[...])
    h = jax.nn.relu(dinv * (p_ref[0] + p_ref[1] + y_ref[...]) + b_ref[...])
    o_ref[...] = dinv * jnp.dot(h, w_ref[...], preferred_element_type=jnp.float32)


def _tc_mid(deg, parts, y, b, w):
    n = y.shape[0]
    grid = (n // _BLK,)
    return pl.pallas_call(
        _tc_mid_kernel,
        grid=grid,
        in_specs=[
            pl.BlockSpec((_BLK, 1), lambda i: (i, 0)),
            pl.BlockSpec((NC, _BLK, D), lambda i: (0, i, 0)),
            pl.BlockSpec((_BLK, D), lambda i: (i, 0)),
            pl.BlockSpec((1, D), lambda i: (0, 0)),
            pl.BlockSpec((D, D), lambda i: (0, 0)),
        ],
        out_specs=pl.BlockSpec((_BLK, D), lambda i: (i, 0)),
        out_shape=jax.ShapeDtypeStruct((n, D), jnp.float32),
    )(deg, parts, y, b, w)


def _tc_final_kernel(deg_ref, p_ref, y_ref, b_ref, o_ref):
    dinv = lax.rsqrt(deg_ref[...])
    o_ref[...] = jax.nn.relu(dinv * (p_ref[0] + p_ref[1] + y_ref[...]) + b_ref[...])


def _tc_final(deg, parts, y, b):
    n = y.shape[0]
    grid = (n // _BLK,)
    return pl.pallas_call(
        _tc_final_kernel,
        grid=grid,
        in_specs=[
            pl.BlockSpec((_BLK, 1), lambda i: (i, 0)),
            pl.BlockSpec((NC, _BLK, D), lambda i: (0, i, 0)),
            pl.BlockSpec((_BLK, D), lambda i: (i, 0)),
            pl.BlockSpec((1, D), lambda i: (0, 0)),
        ],
        out_specs=pl.BlockSpec((_BLK, D), lambda i: (i, 0)),
        out_shape=jax.ShapeDtypeStruct((n, D), jnp.float32),
    )(deg, parts, y, b)


def _tc_deg_kernel(h_ref, o_ref):
    o_ref[...] = h_ref[0, :, 0:1] + h_ref[1, :, 0:1] + 1.0


def _tc_deg(hist):
    n = hist.shape[1]
    grid = (n // _BLK,)
    return pl.pallas_call(
        _tc_deg_kernel,
        grid=grid,
        in_specs=[pl.BlockSpec((NC, _BLK, 16), lambda i: (0, i, 0))],
        out_specs=pl.BlockSpec((_BLK, 1), lambda i: (i, 0)),
        out_shape=jax.ShapeDtypeStruct((n, 1), jnp.float32),
    )(hist)


def kernel(x, edge_index, W1, b1, W2, b2):
    src = edge_index[0].astype(jnp.int32)
    dst = edge_index[1].astype(jnp.int32)

    pad = E_PAD - N_EDGES
    pad_i = jnp.arange(pad, dtype=jnp.int32)
    src_p = jnp.concatenate([src, pad_i % 16]).reshape(N_CHUNKS, K)
    dst_p = jnp.concatenate([dst, N_NODES + (pad_i % 16)]).reshape(N_CHUNKS, K)

    x_p = jnp.zeros((N_PAD, D), jnp.float32).at[:N_NODES].set(x)

    sc_hist = _make_sc_hist()
    sc_agg = _make_sc_agg(D)

    hist = sc_hist(dst_p)
    deg = _tc_deg(hist)

    y1 = _tc_scale_matmul(deg, x_p, W1)
    p1 = sc_agg(y1, src_p, dst_p)
    y2 = _tc_mid(deg, p1, y1, b1.reshape(1, D), W2)
    p2 = sc_agg(y2, src_p, dst_p)
    out = _tc_final(deg, p2, y2, b2.reshape(1, D))
    return out[:N_NODES]
